# Initial kernel scaffold; baseline (speedup 1.0000x reference)
import jax
import jax.numpy as jnp
from jax.experimental import pallas as pl


def kernel(x_drug, x_prot, edge_label_index):
    raise NotImplementedError("write your pallas kernel here")


# speedup vs baseline: 4.2482x; 4.2482x over previous
import functools

import jax
import jax.numpy as jnp
from jax import lax
from jax.experimental import pallas as pl
from jax.experimental.pallas import tpu as pltpu
from jax.experimental.pallas import tpu_sc as plsc

NC = 2
NS = 16
NW = NC * NS

N_NODES = 10000
D = 128
W = D // 2
E_TOTAL = 320000
E_PER_W = E_TOTAL // NW
CHUNK = 80
N_CHUNKS = E_PER_W // CHUNK
HIMASK = -65536


def _sc_body(xd_hbm, xp_hbm, idd_hbm, idp_hbm, out_hbm,
             idd_v, idp_v, rows_a, rows_b, out_v, sem_a, sem_b):
  wid = lax.axis_index("s") * NC + lax.axis_index("c")
  base_w = wid * E_PER_W

  lane = lax.iota(jnp.int32, 16)

  def chunk_body(k, carry):
    base = base_w + k * CHUNK
    pltpu.sync_copy(idd_hbm.at[pl.ds(base, CHUNK)], idd_v)
    pltpu.sync_copy(idp_hbm.at[pl.ds(base, CHUNK)], idp_v)
    cp_a = pltpu.async_copy(xd_hbm.at[idd_v], rows_a, sem_a)
    cp_b = pltpu.async_copy(xp_hbm.at[idp_v], rows_b, sem_b)
    cp_a.wait()
    cp_b.wait()

    def group_body(g, c):
      e0 = g * 16
      res = jnp.zeros((16,), jnp.float32)
      for i in range(16):
        e = e0 + i
        acc = jnp.zeros((16,), jnp.float32)
        for j in range(W // 16):
          wa = rows_a[e, pl.ds(j * 16, 16)]
          wb = rows_b[e, pl.ds(j * 16, 16)]
          alo = lax.bitcast_convert_type(lax.shift_left(wa, 16), jnp.float32)
          ahi = lax.bitcast_convert_type(wa & HIMASK, jnp.float32)
          blo = lax.bitcast_convert_type(lax.shift_left(wb, 16), jnp.float32)
          bhi = lax.bitcast_convert_type(wb & HIMASK, jnp.float32)
          acc = acc + alo * blo
          acc = acc + ahi * bhi
        res = jnp.where(lane == i, jnp.sum(acc), res)
      out_v[pl.ds(e0, 16)] = res
      return c

    lax.fori_loop(0, CHUNK // 16, group_body, 0)
    pltpu.sync_copy(out_v, out_hbm.at[pl.ds(base, CHUNK)])
    return carry

  lax.fori_loop(0, N_CHUNKS, chunk_body, 0)


@functools.partial(jax.jit, static_argnames=("interpret",))
def _run(xd_w, xp_w, idd, idp, interpret=False):
  mesh = plsc.VectorSubcoreMesh(core_axis_name="c", subcore_axis_name="s",
                                num_cores=NC, num_subcores=NS)
  return pl.kernel(
      _sc_body,
      out_type=jax.ShapeDtypeStruct((E_TOTAL,), jnp.float32),
      mesh=mesh,
      scratch_types=[
          pltpu.VMEM((CHUNK,), jnp.int32),
          pltpu.VMEM((CHUNK,), jnp.int32),
          pltpu.VMEM((CHUNK, W), jnp.int32),
          pltpu.VMEM((CHUNK, W), jnp.int32),
          pltpu.VMEM((CHUNK,), jnp.float32),
          pltpu.SemaphoreType.DMA,
          pltpu.SemaphoreType.DMA,
      ],
      compiler_params=pltpu.CompilerParams(needs_layout_passes=False, use_tc_tiling_on_sc=False),
      interpret=interpret,
  )(xd_w, xp_w, idd, idp)


def kernel(x_drug, x_prot, edge_label_index):
  eli = edge_label_index.astype(jnp.int32)
  xd_w = lax.bitcast_convert_type(
      x_drug.astype(jnp.bfloat16).reshape(N_NODES, W, 2), jnp.int32)
  xp_w = lax.bitcast_convert_type(
      x_prot.astype(jnp.bfloat16).reshape(N_NODES, W, 2), jnp.int32)
  return _run(xd_w, xp_w, eli[0], eli[1])


# speedup vs baseline: 4.9983x; 1.1766x over previous
import functools

import jax
import jax.numpy as jnp
from jax import lax
from jax.experimental import pallas as pl
from jax.experimental.pallas import tpu as pltpu
from jax.experimental.pallas import tpu_sc as plsc

NC = 2
NS = 16
NW = NC * NS

N_NODES = 10000
D = 128
W = D // 2
E_TOTAL = 320000
E_PER_W = E_TOTAL // NW
CHUNK = 80
N_CHUNKS = E_PER_W // CHUNK
HIMASK = -65536


def _sc_body(xd_hbm, xp_hbm, idd_hbm, idp_hbm, out_hbm,
             idd_v, idp_v, rows_a, rows_b, out_v, sem_a, sem_b):
  wid = lax.axis_index("s") * NC + lax.axis_index("c")
  base_w = wid * E_PER_W

  lane = lax.iota(jnp.int32, 16)

  def chunk_body(k, carry):
    base = base_w + k * CHUNK
    pltpu.sync_copy(idd_hbm.at[pl.ds(base, CHUNK)], idd_v)
    pltpu.sync_copy(idp_hbm.at[pl.ds(base, CHUNK)], idp_v)
    cp_a = pltpu.async_copy(xd_hbm.at[idd_v], rows_a, sem_a)
    cp_b = pltpu.async_copy(xp_hbm.at[idp_v], rows_b, sem_b)
    cp_a.wait()
    cp_b.wait()

    def group_body(g, c):
      e0 = g * 16
      res = jnp.zeros((16,), jnp.float32)
      for i in range(16):
        e = e0 + i
        acc = jnp.zeros((16,), jnp.float32)
        for j in range(D // 32):
          wa = rows_a[e, pl.ds(j * 32, 32)]
          wb = rows_b[e, pl.ds(j * 32, 32)]
          p0, p1 = plsc.unpack(wa * wb, format=plsc.PackFormat.INTERLEAVED)
          acc = acc + p0
          acc = acc + p1
        res = jnp.where(lane == i, jnp.sum(acc), res)
      out_v[pl.ds(e0, 16)] = res
      return c

    lax.fori_loop(0, CHUNK // 16, group_body, 0)
    pltpu.sync_copy(out_v, out_hbm.at[pl.ds(base, CHUNK)])
    return carry

  lax.fori_loop(0, N_CHUNKS, chunk_body, 0)


@functools.partial(jax.jit, static_argnames=("interpret",))
def _run(xd_w, xp_w, idd, idp, interpret=False):
  mesh = plsc.VectorSubcoreMesh(core_axis_name="c", subcore_axis_name="s",
                                num_cores=NC, num_subcores=NS)
  return pl.kernel(
      _sc_body,
      out_type=jax.ShapeDtypeStruct((E_TOTAL,), jnp.float32),
      mesh=mesh,
      scratch_types=[
          pltpu.VMEM((CHUNK,), jnp.int32),
          pltpu.VMEM((CHUNK,), jnp.int32),
          pltpu.VMEM((CHUNK, D), jnp.bfloat16),
          pltpu.VMEM((CHUNK, D), jnp.bfloat16),
          pltpu.VMEM((CHUNK,), jnp.float32),
          pltpu.SemaphoreType.DMA,
          pltpu.SemaphoreType.DMA,
      ],
      compiler_params=pltpu.CompilerParams(needs_layout_passes=False, use_tc_tiling_on_sc=False),
      interpret=interpret,
  )(xd_w, xp_w, idd, idp)


def kernel(x_drug, x_prot, edge_label_index):
  eli = edge_label_index.astype(jnp.int32)
  return _run(x_drug.astype(jnp.bfloat16), x_prot.astype(jnp.bfloat16),
              eli[0], eli[1])
